# baseline (speedup 1.0000x reference)
import functools
import jax
import jax.numpy as jnp
from jax import lax
from jax.experimental import pallas as pl
from jax.experimental.pallas import tpu as pltpu
from jax.experimental.pallas import tpu_sc as plsc

D_MODEL = 64
SCALE = 8.0
NUM_WORKERS = 32
STEP = 128
STEPS_PER_CHUNK = 8
CHUNK = STEP * STEPS_PER_CHUNK


@functools.partial(jax.jit, static_argnames=("num_steps",))
def _gather_scale(lut, idx2d, *, num_steps):
    total_rows = idx2d.shape[0] * STEP
    steps_per_worker = idx2d.shape[0] // NUM_WORKERS
    chunks_per_worker = steps_per_worker // STEPS_PER_CHUNK
    mesh = plsc.VectorSubcoreMesh(core_axis_name="c", subcore_axis_name="s")

    @functools.partial(
        pl.kernel,
        out_type=jax.ShapeDtypeStruct((total_rows, D_MODEL), jnp.float32),
        mesh=mesh,
        scratch_types=[
            pltpu.VMEM((STEPS_PER_CHUNK, STEP), jnp.int32),
            pltpu.VMEM((CHUNK, D_MODEL), jnp.float32),
            pltpu.SemaphoreType.DMA,
        ],
        compiler_params=pltpu.CompilerParams(use_tc_tiling_on_sc=False),
    )
    def k(lut_hbm, idx_hbm, out_hbm, idx_v, rows_v, sem):
        wid = lax.axis_index("s") * 2 + lax.axis_index("c")
        step0 = wid * steps_per_worker

        def chunk_body(ci, carry):
            sbase = step0 + ci * STEPS_PER_CHUNK
            pltpu.sync_copy(idx_hbm.at[pl.ds(sbase, STEPS_PER_CHUNK)], idx_v)
            copies = [
                pltpu.async_copy(
                    lut_hbm.at[idx_v.at[j]],
                    rows_v.at[pl.ds(j * STEP, STEP)],
                    sem,
                )
                for j in range(STEPS_PER_CHUNK)
            ]
            for cp in copies:
                cp.wait()

            def scale_body(r, c2):
                for j in range(D_MODEL // 16):
                    sl = pl.ds(j * 16, 16)
                    rows_v[r, sl] = rows_v[r, sl] * SCALE
                return c2

            lax.fori_loop(0, CHUNK, scale_body, 0)
            pltpu.sync_copy(rows_v, out_hbm.at[pl.ds(sbase * STEP, CHUNK)])
            return carry

        lax.fori_loop(0, chunks_per_worker, chunk_body, 0)

    return k(lut, idx2d)


def kernel(x, lut):
    b0, b1 = x.shape
    total = b0 * b1
    assert total % (NUM_WORKERS * CHUNK) == 0
    idx2d = x.reshape(total // STEP, STEP).astype(jnp.int32)
    out = _gather_scale(lut, idx2d, num_steps=total // STEP)
    return out.reshape(b0, b1, D_MODEL)


# speedup vs baseline: 1.1085x; 1.1085x over previous
import functools
import jax
import jax.numpy as jnp
from jax import lax
from jax.experimental import pallas as pl
from jax.experimental.pallas import tpu as pltpu
from jax.experimental.pallas import tpu_sc as plsc

D_MODEL = 64
SCALE = 8.0
NUM_WORKERS = 32
STEP = 128
STEPS_PER_CHUNK = 2
CHUNK = STEP * STEPS_PER_CHUNK
NBUF = 4


@jax.jit
def _gather_scale(lut, idx2d):
    num_steps = idx2d.shape[0]
    total_rows = num_steps * STEP
    steps_per_worker = num_steps // NUM_WORKERS
    nch = steps_per_worker // STEPS_PER_CHUNK
    assert nch % NBUF == 0
    mesh = plsc.VectorSubcoreMesh(core_axis_name="c", subcore_axis_name="s")

    @functools.partial(
        pl.kernel,
        out_type=jax.ShapeDtypeStruct((total_rows, D_MODEL), jnp.float32),
        mesh=mesh,
        scratch_types=[
            pltpu.VMEM((steps_per_worker, STEP), jnp.int32),
            [pltpu.VMEM((CHUNK, D_MODEL), jnp.float32) for _ in range(NBUF)],
            [pltpu.SemaphoreType.DMA for _ in range(NBUF)],
            [pltpu.SemaphoreType.DMA for _ in range(NBUF)],
        ],
        compiler_params=pltpu.CompilerParams(use_tc_tiling_on_sc=False),
    )
    def k(lut_hbm, idx_hbm, out_hbm, idx_all, bufs, gsems, osems):
        wid = lax.axis_index("s") * 2 + lax.axis_index("c")
        step0 = wid * steps_per_worker
        row0 = step0 * STEP

        pltpu.sync_copy(idx_hbm.at[pl.ds(step0, steps_per_worker)], idx_all)

        def fire_gather(c, b):
            for j in range(STEPS_PER_CHUNK):
                pltpu.async_copy(
                    lut_hbm.at[idx_all.at[c * STEPS_PER_CHUNK + j]],
                    bufs[b].at[pl.ds(j * STEP, STEP)],
                    gsems[b],
                )

        def drain_gather(c, b):
            for j in range(STEPS_PER_CHUNK):
                pltpu.make_async_copy(
                    lut_hbm.at[idx_all.at[c * STEPS_PER_CHUNK + j]],
                    bufs[b].at[pl.ds(j * STEP, STEP)],
                    gsems[b],
                ).wait()

        PF = NBUF - 2

        for b in range(PF):
            fire_gather(b, b)

        def body(i, carry):
            for b in range(NBUF):
                c = i * NBUF + b
                b_pre = (b + PF) % NBUF

                @pl.when(c + PF <= nch - 1)
                def _():
                    @pl.when(c >= PF)
                    def _():
                        pltpu.make_async_copy(
                            bufs[b_pre],
                            out_hbm.at[pl.ds(row0, CHUNK)],
                            osems[b_pre],
                        ).wait()

                    fire_gather(c + PF, b_pre)

                drain_gather(c, b)

                buf = bufs[b]

                def scale_body(r, c2):
                    for rr in range(4):
                        for j in range(D_MODEL // 16):
                            sl = pl.ds(j * 16, 16)
                            buf[r * 4 + rr, sl] = buf[r * 4 + rr, sl] * SCALE
                    return c2

                lax.fori_loop(0, CHUNK // 4, scale_body, 0, unroll=2)

                pltpu.async_copy(
                    buf,
                    out_hbm.at[pl.ds(row0 + c * CHUNK, CHUNK)],
                    osems[b],
                )
            return carry

        lax.fori_loop(0, nch // NBUF, body, 0)

        for b in range(NBUF):
            pltpu.make_async_copy(
                bufs[b], out_hbm.at[pl.ds(row0, CHUNK)], osems[b]
            ).wait()

    return k(lut, idx2d)


def kernel(x, lut):
    b0, b1 = x.shape
    total = b0 * b1
    assert total % (NUM_WORKERS * CHUNK * NBUF) == 0
    idx2d = x.reshape(total // STEP, STEP).astype(jnp.int32)
    out = _gather_scale(lut, idx2d)
    return out.reshape(b0, b1, D_MODEL)
